# baseline (speedup 1.0000x reference)
import functools

import jax
import jax.numpy as jnp
from jax import lax
from jax.experimental import pallas as pl
from jax.experimental.pallas import tpu as pltpu
from jax.experimental.pallas import tpu_sc as plsc

_ORDERS = (2, 3, 4)


def _prep_body(ids_ref, idx_ref, *, buckets):
    ids = ids_ref[...]
    h = ids
    nh = idx_ref.shape[0]
    b, s = ids.shape
    for i in range(nh):
        order = _ORDERS[i]
        while_shift = order - 1
        start = 1 if i == 0 else _ORDERS[i - 1]
        for j in range(start, order):
            shifted = jnp.concatenate(
                [jnp.zeros((b, j), jnp.int32), ids[:, : s - j]], axis=1
            )
            h = jnp.bitwise_xor(h, shifted)
        idx_ref[i, :, :] = jnp.bitwise_and(h, buckets - 1) + i * buckets


def _prep(ids, num_heads, buckets):
    b, s = ids.shape
    return pl.pallas_call(
        functools.partial(_prep_body, buckets=buckets),
        out_shape=jax.ShapeDtypeStruct((num_heads, b, s), jnp.int32),
    )(ids)


def _make_sc_gather(num_rows, hash_dim, num_idx, dtype):
    info = plsc.get_sparse_core_info()
    nc, ns = info.num_cores, info.num_subcores
    nw = nc * ns
    per_w = num_idx // nw
    assert per_w * nw == num_idx
    chunk = 128
    n_chunks = per_w // chunk
    assert n_chunks * chunk == per_w

    mesh = plsc.VectorSubcoreMesh(core_axis_name="c", subcore_axis_name="s")

    @functools.partial(
        pl.kernel,
        mesh=mesh,
        out_type=jax.ShapeDtypeStruct((num_idx, hash_dim), dtype),
        scratch_types=[
            pltpu.VMEM((per_w,), jnp.int32),
            pltpu.VMEM((2, chunk, hash_dim), dtype),
            pltpu.SemaphoreType.DMA,
            pltpu.SemaphoreType.DMA,
            pltpu.SemaphoreType.DMA,
            pltpu.SemaphoreType.DMA,
        ],
    )
    def gather_k(table_hbm, idx_hbm, out_hbm, idx_v, rows_v, g0, g1, o0, o1):
        wid = lax.axis_index("s") * nc + lax.axis_index("c")
        base = wid * per_w
        gsem = (g0, g1)
        osem = (o0, o1)
        pltpu.sync_copy(idx_hbm.at[pl.ds(base, per_w)], idx_v)

        def start_gather(j):
            b = j % 2
            return pltpu.async_copy(
                table_hbm.at[idx_v.at[pl.ds(j * chunk, chunk)]],
                rows_v.at[b],
                gsem[b],
            )

        def start_out(j):
            b = j % 2
            return pltpu.async_copy(
                rows_v.at[b], out_hbm.at[pl.ds(base + j * chunk, chunk)], osem[b]
            )

        ocp = [None, None]
        gcp = [None, None]
        gcp[0] = start_gather(0)
        for j in range(n_chunks):
            b = j % 2
            nb = (j + 1) % 2
            if j + 1 < n_chunks:
                if ocp[nb] is not None:
                    ocp[nb].wait()
                gcp[nb] = start_gather(j + 1)
            gcp[b].wait()
            ocp[b] = start_out(j)
        ocp[(n_chunks - 1) % 2].wait()
        if ocp[n_chunks % 2] is not None:
            ocp[n_chunks % 2].wait()

    return gather_k


def _final_body(hid_ref, parts_ref, wg_ref, bg_ref, wo_ref, bo_ref, out_ref):
    x = hid_ref[...]
    g = jax.nn.sigmoid(
        jnp.dot(x, wg_ref[...], preferred_element_type=jnp.float32) + bg_ref[...]
    ).astype(jnp.bfloat16)
    nh = parts_ref.shape[0]
    hd = parts_ref.shape[2]
    acc = None
    for i in range(nh):
        p = parts_ref[i].astype(jnp.bfloat16)
        gp = p * g[:, i : i + 1]
        contrib = jnp.dot(
            gp, wo_ref[i * hd : (i + 1) * hd, :], preferred_element_type=jnp.float32
        )
        acc = contrib if acc is None else acc + contrib
    out_ref[...] = acc + bo_ref[...]


def _final(hid2, parts3, wg_t, bg, wo_t, bo, blk=1024):
    nh, bs, hd = parts3.shape
    d = wo_t.shape[1]
    return pl.pallas_call(
        _final_body,
        grid=(bs // blk,),
        in_specs=[
            pl.BlockSpec((blk, d), lambda t: (t, 0)),
            pl.BlockSpec((nh, blk, hd), lambda t: (0, t, 0)),
            pl.BlockSpec((d, 8), lambda t: (0, 0)),
            pl.BlockSpec((1, 8), lambda t: (0, 0)),
            pl.BlockSpec(wo_t.shape, lambda t: (0, 0)),
            pl.BlockSpec((1, d), lambda t: (0, 0)),
        ],
        out_specs=pl.BlockSpec((blk, d), lambda t: (t, 0)),
        out_shape=jax.ShapeDtypeStruct((bs, d), jnp.float32),
    )(hid2, parts3, wg_t, bg, wo_t, bo)


def kernel(input_ids, hidden_state, tables, W_gate, b_gate, W_out, b_out):
    b, s = input_ids.shape
    nh, buckets, hd = tables.shape
    d = hidden_state.shape[-1]
    bs = b * s

    ids = input_ids.astype(jnp.int32)
    idx3 = _prep(ids, nh, buckets)
    idx_flat = idx3.reshape(-1)

    tables_flat = tables.reshape(nh * buckets, hd)
    parts = _make_sc_gather(nh * buckets, hd, nh * bs, jnp.float32)(
        tables_flat, idx_flat
    )
    parts3 = parts.reshape(nh, bs, hd)

    hid2 = hidden_state.reshape(bs, d)
    wg_t = jnp.zeros((d, 8), jnp.float32).at[:, :nh].set(W_gate.T)
    bg = jnp.zeros((1, 8), jnp.float32).at[0, :nh].set(b_gate)
    wo_t = W_out.T.astype(jnp.bfloat16)
    bo = b_out.reshape(1, d)

    out = _final(hid2, parts3, wg_t, bg, wo_t, bo)
    return out.reshape(b, s, d)


# speedup vs baseline: 1.0168x; 1.0168x over previous
import functools

import jax
import jax.numpy as jnp
from jax import lax
from jax.experimental import pallas as pl
from jax.experimental.pallas import tpu as pltpu
from jax.experimental.pallas import tpu_sc as plsc

_ORDERS = (2, 3, 4)


def _prep_body(ids_ref, idx_ref, *, buckets):
    ids = ids_ref[...]
    h = ids
    nh = idx_ref.shape[0]
    b, s = ids.shape
    for i in range(nh):
        order = _ORDERS[i]
        start = 1 if i == 0 else _ORDERS[i - 1]
        for j in range(start, order):
            shifted = jnp.concatenate(
                [jnp.zeros((b, j), jnp.int32), ids[:, : s - j]], axis=1
            )
            h = jnp.bitwise_xor(h, shifted)
        idx_ref[i, :, :] = jnp.bitwise_and(h, buckets - 1) + i * buckets


def _prep(ids, num_heads, buckets):
    b, s = ids.shape
    return pl.pallas_call(
        functools.partial(_prep_body, buckets=buckets),
        out_shape=jax.ShapeDtypeStruct((num_heads, b, s), jnp.int32),
    )(ids)


def _make_sc_gather(num_rows, hash_dim, num_idx):
    info = plsc.get_sparse_core_info()
    nc, ns = info.num_cores, info.num_subcores
    nw = nc * ns
    per_w = num_idx // nw
    assert per_w * nw == num_idx
    chunk = 128
    n_chunks = per_w // chunk
    assert n_chunks * chunk == per_w

    mesh = plsc.VectorSubcoreMesh(core_axis_name="c", subcore_axis_name="s")

    @functools.partial(
        pl.kernel,
        mesh=mesh,
        out_type=jax.ShapeDtypeStruct((num_idx, hash_dim), jnp.float32),
        scratch_types=[
            pltpu.VMEM((per_w,), jnp.int32),
            pltpu.VMEM((2, chunk, hash_dim), jnp.float32),
            pltpu.SemaphoreType.DMA,
            pltpu.SemaphoreType.DMA,
            pltpu.SemaphoreType.DMA,
            pltpu.SemaphoreType.DMA,
        ],
    )
    def gather_k(table_hbm, idx_hbm, out_hbm, idx_v, rows_v, g0, g1, o0, o1):
        wid = lax.axis_index("s") * nc + lax.axis_index("c")
        base = wid * per_w
        gsem = (g0, g1)
        osem = (o0, o1)
        pltpu.sync_copy(idx_hbm.at[pl.ds(base, per_w)], idx_v)

        def start_gather(j):
            b = j % 2
            return pltpu.async_copy(
                table_hbm.at[idx_v.at[pl.ds(j * chunk, chunk)]],
                rows_v.at[b],
                gsem[b],
            )

        def start_out(j):
            b = j % 2
            return pltpu.async_copy(
                rows_v.at[b], out_hbm.at[pl.ds(base + j * chunk, chunk)], osem[b]
            )

        ocp = [None, None]
        gcp = [None, None]
        gcp[0] = start_gather(0)
        for j in range(n_chunks):
            b = j % 2
            nb = (j + 1) % 2
            if j + 1 < n_chunks:
                if ocp[nb] is not None:
                    ocp[nb].wait()
                gcp[nb] = start_gather(j + 1)
            gcp[b].wait()
            ocp[b] = start_out(j)
        ocp[(n_chunks - 1) % 2].wait()
        if ocp[n_chunks % 2] is not None:
            ocp[n_chunks % 2].wait()

    return gather_k


def _gates_body(hid_ref, wg_ref, bg_ref, g_ref):
    x = hid_ref[...]
    xwg = lax.dot_general(
        x,
        wg_ref[...],
        dimension_numbers=(((1,), (1,)), ((), ())),
        preferred_element_type=jnp.float32,
    )
    g_ref[...] = jax.nn.sigmoid(xwg + bg_ref[...])


def _gates(hid2, wg, bg, blk=2048):
    bs, d = hid2.shape
    nh = wg.shape[0]
    return pl.pallas_call(
        _gates_body,
        grid=(bs // blk,),
        in_specs=[
            pl.BlockSpec((blk, d), lambda t: (t, 0)),
            pl.BlockSpec((nh, d), lambda t: (0, 0)),
            pl.BlockSpec((1, nh), lambda t: (0, 0)),
        ],
        out_specs=pl.BlockSpec((blk, nh), lambda t: (t, 0)),
        out_shape=jax.ShapeDtypeStruct((bs, nh), jnp.float32),
    )(hid2, wg, bg)


def _final_body(parts_ref, g_ref, wo_ref, bo_ref, out_ref):
    g = g_ref[...].astype(jnp.bfloat16)
    nh = parts_ref.shape[0]
    hd = parts_ref.shape[2]
    acc = None
    for i in range(nh):
        p = parts_ref[i].astype(jnp.bfloat16)
        gp = p * g[:, i : i + 1]
        contrib = lax.dot_general(
            gp,
            wo_ref[:, i * hd : (i + 1) * hd],
            dimension_numbers=(((1,), (1,)), ((), ())),
            preferred_element_type=jnp.float32,
        )
        acc = contrib if acc is None else acc + contrib
    out_ref[...] = acc + bo_ref[...]


def _final(parts3, g, wo_b, bo, blk=1024):
    nh, bs, hd = parts3.shape
    d = wo_b.shape[0]
    return pl.pallas_call(
        _final_body,
        grid=(bs // blk,),
        in_specs=[
            pl.BlockSpec((nh, blk, hd), lambda t: (0, t, 0)),
            pl.BlockSpec((blk, nh), lambda t: (t, 0)),
            pl.BlockSpec(wo_b.shape, lambda t: (0, 0)),
            pl.BlockSpec((1, d), lambda t: (0, 0)),
        ],
        out_specs=pl.BlockSpec((blk, d), lambda t: (t, 0)),
        out_shape=jax.ShapeDtypeStruct((bs, d), jnp.float32),
    )(parts3, g, wo_b, bo)


def kernel(input_ids, hidden_state, tables, W_gate, b_gate, W_out, b_out):
    b, s = input_ids.shape
    nh, buckets, hd = tables.shape
    d = hidden_state.shape[-1]
    bs = b * s

    ids = input_ids.astype(jnp.int32)
    idx3 = _prep(ids, nh, buckets)
    idx_flat = idx3.reshape(-1)

    tables_flat = tables.reshape(nh * buckets, hd)
    parts = _make_sc_gather(nh * buckets, hd, nh * bs)(tables_flat, idx_flat)
    parts3 = parts.reshape(nh, bs, hd)

    hid2 = hidden_state.reshape(bs, d)
    bg = b_gate.reshape(1, nh)
    wo_b = W_out.astype(jnp.bfloat16)
    bo = b_out.reshape(1, d)

    g = _gates(hid2, W_gate, bg)
    out = _final(parts3, g, wo_b, bo)
    return out.reshape(b, s, d)


# speedup vs baseline: 1.0546x; 1.0372x over previous
import functools

import jax
import jax.numpy as jnp
from jax import lax
from jax.experimental import pallas as pl
from jax.experimental.pallas import tpu as pltpu
from jax.experimental.pallas import tpu_sc as plsc

_ORDERS = (2, 3, 4)


def _prep_body(ids_ref, idx_ref, *, buckets, num_heads):
    ids = ids_ref[...]
    h = ids
    b, s = ids.shape
    for i in range(num_heads):
        order = _ORDERS[i]
        start = 1 if i == 0 else _ORDERS[i - 1]
        for j in range(start, order):
            shifted = jnp.concatenate(
                [jnp.zeros((b, j), jnp.int32), ids[:, : s - j]], axis=1
            )
            h = jnp.bitwise_xor(h, shifted)
        hid = jnp.bitwise_and(h, buckets - 1) + i * buckets
        for bb in range(b):
            idx_ref[pl.ds((i * b + bb) * s, s)] = hid[bb]


def _prep(ids, num_heads, buckets):
    b, s = ids.shape
    return pl.pallas_call(
        functools.partial(_prep_body, buckets=buckets, num_heads=num_heads),
        out_shape=jax.ShapeDtypeStruct((num_heads * b * s,), jnp.int32),
    )(ids)


def _make_sc_gather(num_rows, hash_dim, num_idx):
    info = plsc.get_sparse_core_info()
    nc, ns = info.num_cores, info.num_subcores
    nw = nc * ns
    per_w = num_idx // nw
    assert per_w * nw == num_idx
    chunk = 128
    n_chunks = per_w // chunk
    assert n_chunks * chunk == per_w

    mesh = plsc.VectorSubcoreMesh(core_axis_name="c", subcore_axis_name="s")

    @functools.partial(
        pl.kernel,
        mesh=mesh,
        out_type=jax.ShapeDtypeStruct((num_idx, hash_dim), jnp.float32),
        scratch_types=[
            pltpu.VMEM((per_w,), jnp.int32),
            pltpu.VMEM((2, chunk, hash_dim), jnp.float32),
            pltpu.SemaphoreType.DMA,
            pltpu.SemaphoreType.DMA,
            pltpu.SemaphoreType.DMA,
            pltpu.SemaphoreType.DMA,
        ],
    )
    def gather_k(table_hbm, idx_hbm, out_hbm, idx_v, rows_v, g0, g1, o0, o1):
        wid = lax.axis_index("s") * nc + lax.axis_index("c")
        base = wid * per_w
        gsem = (g0, g1)
        osem = (o0, o1)
        pltpu.sync_copy(idx_hbm.at[pl.ds(base, per_w)], idx_v)

        def start_gather(j):
            b = j % 2
            return pltpu.async_copy(
                table_hbm.at[idx_v.at[pl.ds(j * chunk, chunk)]],
                rows_v.at[b],
                gsem[b],
            )

        def start_out(j):
            b = j % 2
            return pltpu.async_copy(
                rows_v.at[b], out_hbm.at[pl.ds(base + j * chunk, chunk)], osem[b]
            )

        ocp = [None, None]
        gcp = [None, None]
        gcp[0] = start_gather(0)
        for j in range(n_chunks):
            b = j % 2
            nb = (j + 1) % 2
            if j + 1 < n_chunks:
                if ocp[nb] is not None:
                    ocp[nb].wait()
                gcp[nb] = start_gather(j + 1)
            gcp[b].wait()
            ocp[b] = start_out(j)
        ocp[(n_chunks - 1) % 2].wait()
        if ocp[n_chunks % 2] is not None:
            ocp[n_chunks % 2].wait()

    return gather_k


def _gates_body(hid_ref, wg_ref, bg_ref, g_ref):
    x = hid_ref[...]
    xwg = lax.dot_general(
        x,
        wg_ref[...],
        dimension_numbers=(((1,), (1,)), ((), ())),
        preferred_element_type=jnp.float32,
    )
    g_ref[...] = jax.nn.sigmoid(xwg + bg_ref[...])


def _gates(hid2, wg, bg, blk=2048):
    bs, d = hid2.shape
    nh = wg.shape[0]
    return pl.pallas_call(
        _gates_body,
        grid=(bs // blk,),
        in_specs=[
            pl.BlockSpec((blk, d), lambda t: (t, 0)),
            pl.BlockSpec((nh, d), lambda t: (0, 0)),
            pl.BlockSpec((1, nh), lambda t: (0, 0)),
        ],
        out_specs=pl.BlockSpec((blk, nh), lambda t: (t, 0)),
        out_shape=jax.ShapeDtypeStruct((bs, nh), jnp.float32),
    )(hid2, wg, bg)


def _final_body(parts_ref, g_ref, wo_ref, bo_ref, out_ref):
    g = g_ref[...].astype(jnp.bfloat16)
    nh = parts_ref.shape[0]
    hd = parts_ref.shape[2]
    acc = None
    for i in range(nh):
        p = parts_ref[i].astype(jnp.bfloat16)
        gp = p * g[:, i : i + 1]
        contrib = lax.dot_general(
            gp,
            wo_ref[:, i * hd : (i + 1) * hd].astype(jnp.bfloat16),
            dimension_numbers=(((1,), (1,)), ((), ())),
            preferred_element_type=jnp.float32,
        )
        acc = contrib if acc is None else acc + contrib
    out_ref[...] = acc + bo_ref[...]


def _final(parts3, g, wo_b, bo, blk=1024):
    nh, bs, hd = parts3.shape
    d = wo_b.shape[0]
    return pl.pallas_call(
        _final_body,
        grid=(bs // blk,),
        in_specs=[
            pl.BlockSpec((nh, blk, hd), lambda t: (0, t, 0)),
            pl.BlockSpec((blk, nh), lambda t: (t, 0)),
            pl.BlockSpec(wo_b.shape, lambda t: (0, 0)),
            pl.BlockSpec((1, d), lambda t: (0, 0)),
        ],
        out_specs=pl.BlockSpec((blk, d), lambda t: (t, 0)),
        out_shape=jax.ShapeDtypeStruct((bs, d), jnp.float32),
    )(parts3, g, wo_b, bo)


def kernel(input_ids, hidden_state, tables, W_gate, b_gate, W_out, b_out):
    b, s = input_ids.shape
    nh, buckets, hd = tables.shape
    d = hidden_state.shape[-1]
    bs = b * s

    ids = input_ids.astype(jnp.int32)
    idx_flat = _prep(ids, nh, buckets)

    tables_flat = tables.reshape(nh * buckets, hd)
    parts = _make_sc_gather(nh * buckets, hd, nh * bs)(tables_flat, idx_flat)
    parts3 = parts.reshape(nh, bs, hd)

    hid2 = hidden_state.reshape(bs, d)
    bg = b_gate.reshape(1, nh)
    bo = b_out.reshape(1, d)

    g = _gates(hid2, W_gate, bg)
    out = _final(parts3, g, W_out, bo)
    return out.reshape(b, s, d)
